# baseline (speedup 1.0000x reference)
import functools
import math

import jax
import jax.numpy as jnp
from jax import lax
from jax.experimental import pallas as pl
from jax.experimental.pallas import tpu as pltpu
from jax.experimental.pallas import tpu_sc as plsc

_EPS = 1e-5
_RADIUS = 0.2
_K = 16
_BIG16 = 32000


def _ballquery_body(prT_ref, pcT_ref, out_ref):
    b = pl.program_id(0)
    prT = prT_ref[0]
    pcT = pcT_ref[0]
    n = pcT.shape[0]
    rb = prT.shape[1]

    rowsq = (prT[0:1, :] * prT[0:1, :]
             + prT[1:2, :] * prT[1:2, :]
             + prT[2:3, :] * prT[2:3, :])
    colsq = (pcT[:, 0:1] * pcT[:, 0:1]
             + pcT[:, 1:2] * pcT[:, 1:2]
             + pcT[:, 2:3] * pcT[:, 2:3])
    cross = jnp.dot(pcT.astype(jnp.bfloat16), prT.astype(jnp.bfloat16),
                    preferred_element_type=jnp.float32)
    sqr = colsq + rowsq - 2.0 * cross

    col = lax.broadcasted_iota(jnp.int32, (n, rb), 0)
    big = jnp.int32(_BIG16)
    cur = jnp.where(sqr > jnp.float32(_RADIUS * _RADIUS), big, col)

    chk = 128
    iota16 = lax.broadcasted_iota(jnp.int32, (_K, rb), 0)
    cnt = jnp.zeros((rb,), jnp.int32)
    out = jnp.full((_K, rb), jnp.int32(n))

    def body(carry):
        cnt, out, chunk, _ = carry
        m = jnp.min(chunk, axis=0)
        valid = (m < big) & (cnt < _K)
        sel = (iota16 == cnt[None, :]) & valid[None, :]
        out2 = jnp.where(sel, m[None, :], out)
        cnt2 = cnt + valid.astype(jnp.int32)
        chunk2 = jnp.where(chunk <= m[None, :], big, chunk)
        return (cnt2, out2, chunk2, jnp.any(valid))

    for c in range(n // chk):
        chunk = cur[c * chk:(c + 1) * chk, :]
        go0 = jnp.any(cnt < _K)
        cnt, out, _, _ = lax.while_loop(
            lambda carry: carry[3], body, (cnt, out, chunk, go0))

    filled = iota16 < cnt[None, :]
    res = jnp.where(filled, out, out[0:1, :])
    out_ref[0] = res + b * n


def _ball_query(input_p, pT, rb):
    B, _, N = input_p.shape
    grid = (B, N // rb)
    return pl.pallas_call(
        _ballquery_body,
        grid=grid,
        in_specs=[
            pl.BlockSpec((1, 3, rb), lambda b, i: (b, 0, i)),
            pl.BlockSpec((1, N, 3), lambda b, i: (b, 0, 0)),
        ],
        out_specs=pl.BlockSpec((1, _K, rb), lambda b, i: (b, 0, i)),
        out_shape=jax.ShapeDtypeStruct((B, _K, N), jnp.int32),
    )(input_p, pT)


def _sc_gather(xtab, ptab, idx_flat):
    G = idx_flat.shape[0]
    Dx = xtab.shape[1]
    Dp = ptab.shape[1]
    info = plsc.get_sparse_core_info()
    NC, NS = info.num_cores, info.num_subcores
    NW = NC * NS
    CH = 512
    NBUF = 2
    nch = G // (NW * CH)
    mesh = plsc.VectorSubcoreMesh(core_axis_name="c", subcore_axis_name="s")

    @functools.partial(
        pl.kernel,
        out_type=(
            jax.ShapeDtypeStruct((G, Dx), jnp.float32),
            jax.ShapeDtypeStruct((G, Dp), jnp.float32),
        ),
        mesh=mesh,
        compiler_params=pltpu.CompilerParams(use_tc_tiling_on_sc=False),
        scratch_types=[
            [pltpu.VMEM((CH,), jnp.int32) for _ in range(NBUF)],
            [pltpu.VMEM((CH, Dx), jnp.float32) for _ in range(NBUF)],
            [pltpu.VMEM((CH, Dp), jnp.float32) for _ in range(NBUF)],
            [pltpu.SemaphoreType.DMA for _ in range(NBUF)],
            [pltpu.SemaphoreType.DMA for _ in range(NBUF)],
            [pltpu.SemaphoreType.DMA for _ in range(NBUF)],
            [pltpu.SemaphoreType.DMA for _ in range(NBUF)],
        ],
    )
    def gather_k(xtab_hbm, ptab_hbm, idx_hbm, outx_hbm, outp_hbm,
                 idx_v, rx_v, rp_v, semi, semx, semp, semo):
        wid = lax.axis_index("s") * NC + lax.axis_index("c")

        def start(c, slot):
            base = (wid * nch + c) * CH
            pltpu.async_copy(idx_hbm.at[pl.ds(base, CH)], idx_v[slot],
                             semi[slot]).wait()
            pltpu.async_copy(xtab_hbm.at[idx_v[slot]], rx_v[slot], semx[slot])
            pltpu.async_copy(ptab_hbm.at[idx_v[slot]], rp_v[slot], semp[slot])

        def drain(c, slot):
            base = (wid * nch + c) * CH
            pltpu.make_async_copy(xtab_hbm.at[idx_v[slot]], rx_v[slot],
                                  semx[slot]).wait()
            pltpu.make_async_copy(ptab_hbm.at[idx_v[slot]], rp_v[slot],
                                  semp[slot]).wait()
            pltpu.async_copy(rx_v[slot], outx_hbm.at[pl.ds(base, CH)],
                             semo[slot])
            pltpu.async_copy(rp_v[slot], outp_hbm.at[pl.ds(base, CH)],
                             semo[slot])

        def wait_out(c, slot):
            base = (wid * nch + c) * CH
            pltpu.make_async_copy(rx_v[slot], outx_hbm.at[pl.ds(base, CH)],
                                  semo[slot]).wait()
            pltpu.make_async_copy(rp_v[slot], outp_hbm.at[pl.ds(base, CH)],
                                  semo[slot]).wait()

        for s in range(NBUF):
            start(s, s)
        for c in range(nch):
            slot = c % NBUF
            drain(c, slot)
            if c + NBUF < nch:
                wait_out(c, slot)
                start(c + NBUF, slot)
            else:
                wait_out(c, slot)

    return gather_k(xtab, ptab, idx_flat)


def _mlp_body(pr_ref, xres_ref, gx_ref, gp_ref, vt_ref, c1_ref, w2_ref, c2_ref,
              a1_ref, ca1_ref, a2_ref, ca2_ref, ld_ref, cl_ref, out_ref):
    pr = pr_ref[0]
    gx = gx_ref[0]
    gp = gp_ref[0]
    kk, rb, c = gx.shape

    acc = None
    for d in range(3):
        rel_d = pr[None, :, d:d + 1] - gp[:, :, d:d + 1]
        term = rel_d * vt_ref[d:d + 1, :][None]
        acc = term if acc is None else acc + term
    bf = jnp.bfloat16
    pe1 = jnp.maximum(acc + c1_ref[0][None, None, :], 0.0)
    pe1 = pe1.reshape(kk * rb, c)
    pe = jnp.dot(pe1.astype(bf), w2_ref[...].astype(bf),
                 preferred_element_type=jnp.float32)
    pe = pe + c2_ref[0][None, :]
    h = gx.reshape(kk * rb, c) + pe
    a1 = jnp.maximum(
        jnp.dot(h.astype(bf), a1_ref[...].astype(bf),
                preferred_element_type=jnp.float32)
        + ca1_ref[0][None, :], 0.0)
    a2 = jnp.maximum(
        jnp.dot(a1.astype(bf), a2_ref[...].astype(bf),
                preferred_element_type=jnp.float32)
        + ca2_ref[0][None, :], 0.0)
    y = jnp.max(a2.reshape(kk, rb, c), axis=0)
    out = jnp.dot(y.astype(bf), ld_ref[...].astype(bf),
                  preferred_element_type=jnp.float32)
    out_ref[0] = out + cl_ref[0][None, :] + xres_ref[0]


def _mlp(pT, xT, gx, gp, weights, rb):
    B, N, C = xT.shape
    kk = _K
    vt, c1, w2, c2, a1, ca1, a2, ca2, ld, cl = weights
    grid = (B, N // rb)
    wspec = lambda shp: pl.BlockSpec(shp, lambda b, i: tuple(0 for _ in shp))
    return pl.pallas_call(
        _mlp_body,
        grid=grid,
        in_specs=[
            pl.BlockSpec((1, rb, 3), lambda b, i: (b, i, 0)),
            pl.BlockSpec((1, rb, C), lambda b, i: (b, i, 0)),
            pl.BlockSpec((1, kk, rb, C), lambda b, i: (b, 0, i, 0)),
            pl.BlockSpec((1, kk, rb, 16), lambda b, i: (b, 0, i, 0)),
            wspec((3, C)), wspec((1, C)),
            wspec((C, C)), wspec((1, C)),
            wspec((C, C)), wspec((1, C)),
            wspec((C, C)), wspec((1, C)),
            wspec((C, C)), wspec((1, C)),
        ],
        out_specs=pl.BlockSpec((1, rb, C), lambda b, i: (b, i, 0)),
        out_shape=jax.ShapeDtypeStruct((B, N, C), jnp.float32),
    )(pT, xT, gx, gp, vt, c1, w2, c2, a1, ca1, a2, ca2, ld, cl)


def kernel(input_p, input_x, dW1, db1, dg1, dbt1, dW2, db2, dg2, dbt2,
           aW1, ab1, ag1, abt1, aW2, ab2, ag2, abt2, ldW, ldb, ldg, ldbt):
    B, C, N = input_x.shape
    scale = jnp.float32(1.0 / math.sqrt(1.0 + _EPS))

    def fold(W, bias, g, bt):
        s = g * scale
        return (W * s[:, None]).T, (bias * s + bt)[None, :]

    vt, c1 = fold(dW1, db1, dg1, dbt1)
    w2, c2 = fold(dW2, db2, dg2, dbt2)
    a1, ca1 = fold(aW1, ab1, ag1, abt1)
    a2, ca2 = fold(aW2, ab2, ag2, abt2)
    ld, cl = fold(ldW, ldb, ldg, ldbt)

    pT = jnp.transpose(input_p, (0, 2, 1))
    xT = jnp.transpose(input_x, (0, 2, 1))

    idxg = _ball_query(input_p, pT, rb=256)

    xtab = xT.reshape(B * N, C)
    ptab = jnp.pad(pT, ((0, 0), (0, 0), (0, 13))).reshape(B * N, 16)
    gx, gp = _sc_gather(xtab, ptab, idxg.reshape(-1))
    gx = gx.reshape(B, _K, N, C)
    gp = gp.reshape(B, _K, N, 16)

    weights = (vt, c1, w2, c2, a1, ca1, a2, ca2, ld, cl)
    y = _mlp(pT, xT, gx, gp, weights, rb=512)
    return (input_p, jnp.transpose(y, (0, 2, 1)))


# speedup vs baseline: 1.5574x; 1.5574x over previous
import functools
import math

import jax
import jax.numpy as jnp
from jax import lax
from jax.experimental import pallas as pl
from jax.experimental.pallas import tpu as pltpu
from jax.experimental.pallas import tpu_sc as plsc

_EPS = 1e-5
_RADIUS = 0.2
_K = 16
_BIG16 = 32000


def _ballquery_body(prT_ref, pcT_ref, out_ref):
    b = pl.program_id(0)
    prT = prT_ref[0]
    pcT = pcT_ref[0]
    n = pcT.shape[0]
    rb = prT.shape[1]

    rowsq = (prT[0:1, :] * prT[0:1, :]
             + prT[1:2, :] * prT[1:2, :]
             + prT[2:3, :] * prT[2:3, :])
    colsq = (pcT[:, 0:1] * pcT[:, 0:1]
             + pcT[:, 1:2] * pcT[:, 1:2]
             + pcT[:, 2:3] * pcT[:, 2:3])
    cross = jnp.dot(pcT.astype(jnp.bfloat16), prT.astype(jnp.bfloat16),
                    preferred_element_type=jnp.float32)
    sqr = colsq + rowsq - 2.0 * cross

    col = lax.broadcasted_iota(jnp.int32, (n, rb), 0)
    cur = jnp.where(sqr > jnp.float32(_RADIUS * _RADIUS), jnp.int32(n), col)

    first = None
    for s in range(_K):
        m = jnp.min(cur, axis=0)
        val = jnp.minimum(m, jnp.int32(n))
        if first is None:
            first = val
            res = val
        else:
            res = jnp.where(val == jnp.int32(n), first, val)
        out_ref[0, s, :] = res + b * n
        if s != _K - 1:
            cur = jnp.where(cur <= m[None, :], jnp.int32(_BIG16), cur)


def _ball_query(input_p, pT, rb):
    B, _, N = input_p.shape
    grid = (B, N // rb)
    return pl.pallas_call(
        _ballquery_body,
        grid=grid,
        in_specs=[
            pl.BlockSpec((1, 3, rb), lambda b, i: (b, 0, i)),
            pl.BlockSpec((1, N, 3), lambda b, i: (b, 0, 0)),
        ],
        out_specs=pl.BlockSpec((1, _K, rb), lambda b, i: (b, 0, i)),
        out_shape=jax.ShapeDtypeStruct((B, _K, N), jnp.int32),
    )(input_p, pT)


def _sc_gather(xtab, ptab, idx_flat):
    G = idx_flat.shape[0]
    Dx = xtab.shape[1]
    Dp = ptab.shape[1]
    info = plsc.get_sparse_core_info()
    NC, NS = info.num_cores, info.num_subcores
    NW = NC * NS
    CH = 512
    NBUF = 2
    nch = G // (NW * CH)
    mesh = plsc.VectorSubcoreMesh(core_axis_name="c", subcore_axis_name="s")

    @functools.partial(
        pl.kernel,
        out_type=(
            jax.ShapeDtypeStruct((G, Dx), jnp.float32),
            jax.ShapeDtypeStruct((G, Dp), jnp.float32),
        ),
        mesh=mesh,
        compiler_params=pltpu.CompilerParams(use_tc_tiling_on_sc=False),
        scratch_types=[
            [pltpu.VMEM((CH,), jnp.int32) for _ in range(NBUF)],
            [pltpu.VMEM((CH, Dx), jnp.float32) for _ in range(NBUF)],
            [pltpu.VMEM((CH, Dp), jnp.float32) for _ in range(NBUF)],
            [pltpu.SemaphoreType.DMA for _ in range(NBUF)],
            [pltpu.SemaphoreType.DMA for _ in range(NBUF)],
            [pltpu.SemaphoreType.DMA for _ in range(NBUF)],
            [pltpu.SemaphoreType.DMA for _ in range(NBUF)],
        ],
    )
    def gather_k(xtab_hbm, ptab_hbm, idx_hbm, outx_hbm, outp_hbm,
                 idx_v, rx_v, rp_v, semi, semx, semp, semo):
        wid = lax.axis_index("s") * NC + lax.axis_index("c")

        def start(c, slot):
            base = (wid * nch + c) * CH
            pltpu.async_copy(idx_hbm.at[pl.ds(base, CH)], idx_v[slot],
                             semi[slot]).wait()
            pltpu.async_copy(xtab_hbm.at[idx_v[slot]], rx_v[slot], semx[slot])
            pltpu.async_copy(ptab_hbm.at[idx_v[slot]], rp_v[slot], semp[slot])

        def drain(c, slot):
            base = (wid * nch + c) * CH
            pltpu.make_async_copy(xtab_hbm.at[idx_v[slot]], rx_v[slot],
                                  semx[slot]).wait()
            pltpu.make_async_copy(ptab_hbm.at[idx_v[slot]], rp_v[slot],
                                  semp[slot]).wait()
            pltpu.async_copy(rx_v[slot], outx_hbm.at[pl.ds(base, CH)],
                             semo[slot])
            pltpu.async_copy(rp_v[slot], outp_hbm.at[pl.ds(base, CH)],
                             semo[slot])

        def wait_out(c, slot):
            base = (wid * nch + c) * CH
            pltpu.make_async_copy(rx_v[slot], outx_hbm.at[pl.ds(base, CH)],
                                  semo[slot]).wait()
            pltpu.make_async_copy(rp_v[slot], outp_hbm.at[pl.ds(base, CH)],
                                  semo[slot]).wait()

        for s in range(NBUF):
            start(s, s)
        for c in range(nch):
            slot = c % NBUF
            drain(c, slot)
            if c + NBUF < nch:
                wait_out(c, slot)
                start(c + NBUF, slot)
            else:
                wait_out(c, slot)

    return gather_k(xtab, ptab, idx_flat)


def _mlp_body(pr_ref, xres_ref, gx_ref, gp_ref, vt_ref, c1_ref, w2_ref, c2_ref,
              a1_ref, ca1_ref, a2_ref, ca2_ref, ld_ref, cl_ref, out_ref):
    pr = pr_ref[0]
    gx = gx_ref[0]
    gp = gp_ref[0]
    kk, rb, c = gx.shape

    acc = None
    for d in range(3):
        rel_d = pr[None, :, d:d + 1] - gp[:, :, d:d + 1]
        term = rel_d * vt_ref[d:d + 1, :][None]
        acc = term if acc is None else acc + term
    bf = jnp.bfloat16
    pe1 = jnp.maximum(acc + c1_ref[0][None, None, :], 0.0)
    pe1 = pe1.reshape(kk * rb, c)
    pe = jnp.dot(pe1.astype(bf), w2_ref[...].astype(bf),
                 preferred_element_type=jnp.float32)
    pe = pe + c2_ref[0][None, :]
    h = gx.reshape(kk * rb, c) + pe
    a1 = jnp.maximum(
        jnp.dot(h.astype(bf), a1_ref[...].astype(bf),
                preferred_element_type=jnp.float32)
        + ca1_ref[0][None, :], 0.0)
    a2 = jnp.maximum(
        jnp.dot(a1.astype(bf), a2_ref[...].astype(bf),
                preferred_element_type=jnp.float32)
        + ca2_ref[0][None, :], 0.0)
    y = jnp.max(a2.reshape(kk, rb, c), axis=0)
    out = jnp.dot(y.astype(bf), ld_ref[...].astype(bf),
                  preferred_element_type=jnp.float32)
    out_ref[0] = out + cl_ref[0][None, :] + xres_ref[0]


def _mlp(pT, xT, gx, gp, weights, rb):
    B, N, C = xT.shape
    kk = _K
    vt, c1, w2, c2, a1, ca1, a2, ca2, ld, cl = weights
    grid = (B, N // rb)
    wspec = lambda shp: pl.BlockSpec(shp, lambda b, i: tuple(0 for _ in shp))
    return pl.pallas_call(
        _mlp_body,
        grid=grid,
        in_specs=[
            pl.BlockSpec((1, rb, 3), lambda b, i: (b, i, 0)),
            pl.BlockSpec((1, rb, C), lambda b, i: (b, i, 0)),
            pl.BlockSpec((1, kk, rb, C), lambda b, i: (b, 0, i, 0)),
            pl.BlockSpec((1, kk, rb, 16), lambda b, i: (b, 0, i, 0)),
            wspec((3, C)), wspec((1, C)),
            wspec((C, C)), wspec((1, C)),
            wspec((C, C)), wspec((1, C)),
            wspec((C, C)), wspec((1, C)),
            wspec((C, C)), wspec((1, C)),
        ],
        out_specs=pl.BlockSpec((1, rb, C), lambda b, i: (b, i, 0)),
        out_shape=jax.ShapeDtypeStruct((B, N, C), jnp.float32),
    )(pT, xT, gx, gp, vt, c1, w2, c2, a1, ca1, a2, ca2, ld, cl)


def kernel(input_p, input_x, dW1, db1, dg1, dbt1, dW2, db2, dg2, dbt2,
           aW1, ab1, ag1, abt1, aW2, ab2, ag2, abt2, ldW, ldb, ldg, ldbt):
    B, C, N = input_x.shape
    scale = jnp.float32(1.0 / math.sqrt(1.0 + _EPS))

    def fold(W, bias, g, bt):
        s = g * scale
        return (W * s[:, None]).T, (bias * s + bt)[None, :]

    vt, c1 = fold(dW1, db1, dg1, dbt1)
    w2, c2 = fold(dW2, db2, dg2, dbt2)
    a1, ca1 = fold(aW1, ab1, ag1, abt1)
    a2, ca2 = fold(aW2, ab2, ag2, abt2)
    ld, cl = fold(ldW, ldb, ldg, ldbt)

    pT = jnp.transpose(input_p, (0, 2, 1))
    xT = jnp.transpose(input_x, (0, 2, 1))

    xtab = xT.reshape(B * N, C)
    ptab = jnp.pad(pT, ((0, 0), (0, 0), (0, 13))).reshape(B * N, 16)
    weights = (vt, c1, w2, c2, a1, ca1, a2, ca2, ld, cl)

    ys = []
    for b in range(B):
        idxg_b = _ball_query(input_p[b:b + 1], pT[b:b + 1], rb=256)
        gx_b, gp_b = _sc_gather(xtab, ptab,
                                (idxg_b + jnp.int32(b * N)).reshape(-1))
        y_b = _mlp(pT[b:b + 1], xT[b:b + 1],
                   gx_b.reshape(1, _K, N, C), gp_b.reshape(1, _K, N, 16),
                   weights, rb=512)
        ys.append(y_b)
    y = jnp.concatenate(ys, axis=0)
    return (input_p, jnp.transpose(y, (0, 2, 1)))
